# baseline (speedup 1.0000x reference)
import functools

import jax
import jax.numpy as jnp
from jax import lax
from jax.experimental import pallas as pl
from jax.experimental.pallas import tpu as pltpu
from jax.experimental.pallas import tpu_sc as plsc

NUM_USERS = 1000000
NUM_ITEMS = 100000
EMB = 64
CONTENT_DIM = 300
BATCH = 16384

_INFO = plsc.get_sparse_core_info()
_NC, _NS = _INFO.num_cores, _INFO.num_subcores
_NW = _NC * _NS
_ROWS_PER_W = BATCH // _NW
_CHUNK = 256
_NCHUNK = _ROWS_PER_W // _CHUNK


def _sc_gather_body(uid_hbm, iid_hbm, nid_hbm, user_hbm, item_hbm, content_hbm,
                    u_out, pos_out, neg_out, c_out,
                    uid_v, iid_v, nid_v, u_v, p_v, n_v, c_v,
                    sem_u, sem_p, sem_n, sem_c):
    wid = lax.axis_index("s") * _NC + lax.axis_index("c")
    for h in range(_NCHUNK):
        base = wid * _ROWS_PER_W + h * _CHUNK
        sl = pl.ds(base, _CHUNK)
        pltpu.sync_copy(uid_hbm.at[sl], uid_v)
        pltpu.sync_copy(iid_hbm.at[sl], iid_v)
        pltpu.sync_copy(nid_hbm.at[sl], nid_v)
        cp_u = pltpu.async_copy(user_hbm.at[uid_v], u_v, sem_u)
        cp_p = pltpu.async_copy(item_hbm.at[iid_v], p_v, sem_p)
        cp_n = pltpu.async_copy(item_hbm.at[nid_v], n_v, sem_n)
        cp_c = pltpu.async_copy(content_hbm.at[iid_v], c_v, sem_c)
        cp_u.wait()
        pltpu.sync_copy(u_v, u_out.at[sl])
        cp_p.wait()
        pltpu.sync_copy(p_v, pos_out.at[sl])
        cp_n.wait()
        pltpu.sync_copy(n_v, neg_out.at[sl])
        cp_c.wait()
        pltpu.sync_copy(c_v, c_out.at[sl])


@jax.jit
def _sc_gather(uid, iid, nid, user_emb, item_emb, item_content):
    mesh = plsc.VectorSubcoreMesh(core_axis_name="c", subcore_axis_name="s")
    fn = pl.kernel(
        _sc_gather_body,
        mesh=mesh,
        out_type=(
            jax.ShapeDtypeStruct((BATCH, EMB), jnp.float32),
            jax.ShapeDtypeStruct((BATCH, EMB), jnp.float32),
            jax.ShapeDtypeStruct((BATCH, EMB), jnp.float32),
            jax.ShapeDtypeStruct((BATCH, CONTENT_DIM), jnp.float32),
        ),
        scratch_types=[
            pltpu.VMEM((_CHUNK,), jnp.int32),
            pltpu.VMEM((_CHUNK,), jnp.int32),
            pltpu.VMEM((_CHUNK,), jnp.int32),
            pltpu.VMEM((_CHUNK, EMB), jnp.float32),
            pltpu.VMEM((_CHUNK, EMB), jnp.float32),
            pltpu.VMEM((_CHUNK, EMB), jnp.float32),
            pltpu.VMEM((_CHUNK, CONTENT_DIM), jnp.float32),
            pltpu.SemaphoreType.DMA,
            pltpu.SemaphoreType.DMA,
            pltpu.SemaphoreType.DMA,
            pltpu.SemaphoreType.DMA,
        ],
        compiler_params=pltpu.CompilerParams(use_tc_tiling_on_sc=False),
    )
    return fn(uid, iid, nid, user_emb, item_emb, item_content)


def _mlp_body(c_ref, w1_ref, b1_ref, w2_ref, b2_ref, out_ref):
    h = jnp.tanh(
        jnp.dot(c_ref[...], w1_ref[...], preferred_element_type=jnp.float32)
        + b1_ref[...]
    )
    out_ref[...] = jnp.tanh(
        jnp.dot(h, w2_ref[...], preferred_element_type=jnp.float32)
        + b2_ref[...]
    )


_BM = 2048


@jax.jit
def _tc_mlp(c, W1, b1, W2, b2):
    b1r = b1.reshape(1, 2 * EMB)
    b2r = b2.reshape(1, EMB)
    return pl.pallas_call(
        _mlp_body,
        grid=(BATCH // _BM,),
        in_specs=[
            pl.BlockSpec((_BM, CONTENT_DIM), lambda i: (i, 0)),
            pl.BlockSpec((CONTENT_DIM, 2 * EMB), lambda i: (0, 0)),
            pl.BlockSpec((1, 2 * EMB), lambda i: (0, 0)),
            pl.BlockSpec((2 * EMB, EMB), lambda i: (0, 0)),
            pl.BlockSpec((1, EMB), lambda i: (0, 0)),
        ],
        out_specs=pl.BlockSpec((_BM, EMB), lambda i: (i, 0)),
        out_shape=jax.ShapeDtypeStruct((BATCH, EMB), jnp.float32),
        compiler_params=pltpu.CompilerParams(
            dimension_semantics=("parallel",),
        ),
    )(c, W1, b1r, W2, b2r)


def kernel(uid, iid, nid, user_emb, item_emb, item_content, W1, b1, W2, b2):
    uid = uid.astype(jnp.int32)
    iid = iid.astype(jnp.int32)
    nid = nid.astype(jnp.int32)
    u, pos, neg, _c_unused = _sc_gather(uid, iid, nid, user_emb, item_emb, item_content)
    c = jnp.take(item_content, iid, axis=0)
    gen = _tc_mlp(c, W1, b1, W2, b2)
    return (u, pos, neg, gen)


# speedup vs baseline: 1.9568x; 1.9568x over previous
import jax
import jax.numpy as jnp
from jax import lax
from jax.experimental import pallas as pl
from jax.experimental.pallas import tpu as pltpu
from jax.experimental.pallas import tpu_sc as plsc

NUM_USERS = 1000000
NUM_ITEMS = 100000
EMB = 64
CONTENT_DIM = 300
BATCH = 16384

_INFO = plsc.get_sparse_core_info()
_NC, _NS = _INFO.num_cores, _INFO.num_subcores
_NW = _NC * _NS
_ROWS_PER_W = BATCH // _NW
_UCHUNK = 256
_NUCHUNK = _ROWS_PER_W // _UCHUNK


_RCHUNK = 512


def _sc_gather_row_body(idx_hbm, tab_hbm, out_hbm, idx_v, row_v, sem):
    wid = lax.axis_index("s") * _NC + lax.axis_index("c")
    base = wid * _ROWS_PER_W
    sl = pl.ds(base, _RCHUNK)
    pltpu.sync_copy(idx_hbm.at[sl], idx_v)
    pltpu.async_copy(tab_hbm.at[idx_v], row_v, sem).wait()
    pltpu.sync_copy(row_v, out_hbm.at[sl])


@jax.jit
def _sc_gather_row(idx, tab):
    mesh = plsc.VectorSubcoreMesh(core_axis_name="c", subcore_axis_name="s")
    fn = pl.kernel(
        _sc_gather_row_body,
        mesh=mesh,
        out_type=jax.ShapeDtypeStruct((BATCH, 2 * EMB), jnp.float32),
        scratch_types=[
            pltpu.VMEM((_RCHUNK,), jnp.int32),
            pltpu.VMEM((_RCHUNK, 2 * EMB), jnp.float32),
            pltpu.SemaphoreType.DMA,
        ],
    )
    return fn(idx, tab)


def _prep_body(ct_ref, it_ref, w1_ref, b1_ref, htab_ref, irm_ref):
    h = lax.dot_general(
        ct_ref[...], w1_ref[...],
        dimension_numbers=(((0,), (0,)), ((), ())),
        preferred_element_type=jnp.float32,
    )
    htab_ref[...] = jnp.tanh(h + b1_ref[...])
    t = jnp.swapaxes(it_ref[...], 0, 1)
    irm_ref[...] = jnp.concatenate(
        [t, jnp.zeros_like(t)], axis=1)


_BN_PREP = 2048


@jax.jit
def _tc_prep(ct, it, W1, b1):
    b1r = b1.reshape(1, 2 * EMB)
    grid = (NUM_ITEMS + _BN_PREP - 1) // _BN_PREP
    return pl.pallas_call(
        _prep_body,
        grid=(grid,),
        in_specs=[
            pl.BlockSpec((CONTENT_DIM, _BN_PREP), lambda i: (0, i)),
            pl.BlockSpec((EMB, _BN_PREP), lambda i: (0, i)),
            pl.BlockSpec((CONTENT_DIM, 2 * EMB), lambda i: (0, 0)),
            pl.BlockSpec((1, 2 * EMB), lambda i: (0, 0)),
        ],
        out_specs=[
            pl.BlockSpec((_BN_PREP, 2 * EMB), lambda i: (i, 0)),
            pl.BlockSpec((_BN_PREP, 2 * EMB), lambda i: (i, 0)),
        ],
        out_shape=[
            jax.ShapeDtypeStruct((NUM_ITEMS, 2 * EMB), jnp.float32),
            jax.ShapeDtypeStruct((NUM_ITEMS, 2 * EMB), jnp.float32),
        ],
        compiler_params=pltpu.CompilerParams(
            dimension_semantics=("parallel",),
        ),
    )(ct, it, W1, b1r)


def _post_body(uw_ref, pw_ref, nw_ref, h_ref, pu_ref, w2_ref, b2_ref,
               u_ref, p_ref, n_ref, g_ref):
    u_ref[...] = jnp.where(pu_ref[...] == 0, uw_ref[:, :EMB], uw_ref[:, EMB:])
    p_ref[...] = pw_ref[:, :EMB]
    n_ref[...] = nw_ref[:, :EMB]
    g_ref[...] = jnp.tanh(
        jnp.dot(h_ref[...], w2_ref[...], preferred_element_type=jnp.float32)
        + b2_ref[...]
    )


_BM_P = 2048


@jax.jit
def _tc_post(uw, pw, nw, h, uid, W2, b2):
    pu = (uid & 1).reshape(BATCH, 1)
    b2r = b2.reshape(1, EMB)
    wide_spec = pl.BlockSpec((_BM_P, 2 * EMB), lambda i: (i, 0))
    out_spec = pl.BlockSpec((_BM_P, EMB), lambda i: (i, 0))
    return pl.pallas_call(
        _post_body,
        grid=(BATCH // _BM_P,),
        in_specs=[
            wide_spec, wide_spec, wide_spec, wide_spec,
            pl.BlockSpec((_BM_P, 1), lambda i: (i, 0)),
            pl.BlockSpec((2 * EMB, EMB), lambda i: (0, 0)),
            pl.BlockSpec((1, EMB), lambda i: (0, 0)),
        ],
        out_specs=[out_spec, out_spec, out_spec, out_spec],
        out_shape=[
            jax.ShapeDtypeStruct((BATCH, EMB), jnp.float32),
            jax.ShapeDtypeStruct((BATCH, EMB), jnp.float32),
            jax.ShapeDtypeStruct((BATCH, EMB), jnp.float32),
            jax.ShapeDtypeStruct((BATCH, EMB), jnp.float32),
        ],
        compiler_params=pltpu.CompilerParams(
            dimension_semantics=("parallel",),
        ),
    )(uw, pw, nw, h, pu, W2, b2r)


def kernel(uid, iid, nid, user_emb, item_emb, item_content, W1, b1, W2, b2):
    uid = uid.astype(jnp.int32)
    iid = iid.astype(jnp.int32)
    nid = nid.astype(jnp.int32)
    it = item_emb.T
    ct = item_content.T
    uw_tab = user_emb.reshape(NUM_USERS // 2, 2 * EMB)
    htab, irm = _tc_prep(ct, it, W1, b1)
    uw = _sc_gather_row(uid >> 1, uw_tab)
    pw = _sc_gather_row(iid, irm)
    nw = _sc_gather_row(nid, irm)
    h = _sc_gather_row(iid, htab)
    u, pos, neg, gen = _tc_post(uw, pw, nw, h, uid, W2, b2)
    return (u, pos, neg, gen)


# speedup vs baseline: 3.2947x; 1.6837x over previous
import jax
import jax.numpy as jnp
from jax import lax
from jax.experimental import pallas as pl
from jax.experimental.pallas import tpu as pltpu
from jax.experimental.pallas import tpu_sc as plsc

NUM_USERS = 1000000
NUM_ITEMS = 100000
EMB = 64
CONTENT_DIM = 300
BATCH = 16384

_INFO = plsc.get_sparse_core_info()
_NC, _NS = _INFO.num_cores, _INFO.num_subcores
_NW = _NC * _NS
_ROWS_PER_W = BATCH // _NW
_UCHUNK = 256
_NUCHUNK = _ROWS_PER_W // _UCHUNK


_NBUF = 8
_UCHUNK = 128
_NUCH = _ROWS_PER_W // _UCHUNK


def _scalar_at(idx_v, j):
    vec = idx_v[pl.ds((j // 16) * 16, 16)]
    mask = lax.iota(jnp.int32, 16) == (j % 16)
    return jnp.sum(jnp.where(mask, vec, 0), axis=0)


def _sc_gatheru_body(uid_hbm, ut_hbm, u_out, idx_v, wbufs, out_v,
                     sem, out_sem):
    wid = lax.axis_index("s") * _NC + lax.axis_index("c")

    def chunk(i, _):
        base = wid * _ROWS_PER_W + i * _UCHUNK
        sl = pl.ds(base, _UCHUNK)
        pltpu.sync_copy(uid_hbm.at[sl], idx_v)
        for g in range(_UCHUNK // _NBUF):
            cps = []
            for k in range(_NBUF):
                j = g * _NBUF + k
                c = _scalar_at(idx_v, j)
                win = lax.shift_right_logical(c, 7)
                off = pl.multiple_of(win * 128, 128)
                cps.append(pltpu.async_copy(
                    ut_hbm.at[:, pl.ds(off, 128)], wbufs.at[k], sem))
            for cp in cps:
                cp.wait()
            for k in range(_NBUF):
                j = g * _NBUF + k
                c = _scalar_at(idx_v, j)
                lane = lax.bitwise_and(c, 127)
                col_idx = jnp.full((16,), lane, jnp.int32)
                slot_idx = jnp.full((16,), k, jnp.int32)
                for m in range(EMB // 16):
                    row_idx = lax.iota(jnp.int32, 16) + (16 * m)
                    gvals = plsc.load_gather(
                        wbufs, [slot_idx, row_idx, col_idx])
                    out_v[j, pl.ds(16 * m, 16)] = gvals
        pltpu.async_copy(out_v, u_out.at[sl], out_sem).wait()

    lax.fori_loop(0, _NUCH, chunk, None)


@jax.jit
def _sc_gatheru(uid, ut):
    mesh = plsc.VectorSubcoreMesh(core_axis_name="c", subcore_axis_name="s")
    fn = pl.kernel(
        _sc_gatheru_body,
        mesh=mesh,
        out_type=jax.ShapeDtypeStruct((BATCH, 2 * EMB), jnp.float32),
        scratch_types=[
            pltpu.VMEM((_UCHUNK,), jnp.int32),
            pltpu.VMEM((_NBUF, EMB, 2 * EMB), jnp.float32),
            pltpu.VMEM((_UCHUNK, 2 * EMB), jnp.float32),
            pltpu.SemaphoreType.DMA,
            pltpu.SemaphoreType.DMA,
        ],
        compiler_params=pltpu.CompilerParams(needs_layout_passes=False),
    )
    return fn(uid, ut)


_RCHUNK = 512


def _sc_gather_row_body(idx_hbm, tab_hbm, out_hbm, idx_v, row_v, sem):
    wid = lax.axis_index("s") * _NC + lax.axis_index("c")
    base = wid * _ROWS_PER_W
    sl = pl.ds(base, _RCHUNK)
    pltpu.sync_copy(idx_hbm.at[sl], idx_v)
    pltpu.async_copy(tab_hbm.at[idx_v], row_v, sem).wait()
    pltpu.sync_copy(row_v, out_hbm.at[sl])


@jax.jit
def _sc_gather_row(idx, tab):
    mesh = plsc.VectorSubcoreMesh(core_axis_name="c", subcore_axis_name="s")
    fn = pl.kernel(
        _sc_gather_row_body,
        mesh=mesh,
        out_type=jax.ShapeDtypeStruct((BATCH, 2 * EMB), jnp.float32),
        scratch_types=[
            pltpu.VMEM((_RCHUNK,), jnp.int32),
            pltpu.VMEM((_RCHUNK, 2 * EMB), jnp.float32),
            pltpu.SemaphoreType.DMA,
        ],
    )
    return fn(idx, tab)


def _prep_body(ct_ref, it_ref, w1_ref, b1_ref, htab_ref, irm_ref):
    h = lax.dot_general(
        ct_ref[...], w1_ref[...],
        dimension_numbers=(((0,), (0,)), ((), ())),
        preferred_element_type=jnp.float32,
    )
    htab_ref[...] = jnp.tanh(h + b1_ref[...])
    t = jnp.swapaxes(it_ref[...], 0, 1)
    irm_ref[...] = jnp.concatenate(
        [t, jnp.zeros_like(t)], axis=1)


_BN_PREP = 2048


@jax.jit
def _tc_prep(ct, it, W1, b1):
    b1r = b1.reshape(1, 2 * EMB)
    grid = (NUM_ITEMS + _BN_PREP - 1) // _BN_PREP
    return pl.pallas_call(
        _prep_body,
        grid=(grid,),
        in_specs=[
            pl.BlockSpec((CONTENT_DIM, _BN_PREP), lambda i: (0, i)),
            pl.BlockSpec((EMB, _BN_PREP), lambda i: (0, i)),
            pl.BlockSpec((CONTENT_DIM, 2 * EMB), lambda i: (0, 0)),
            pl.BlockSpec((1, 2 * EMB), lambda i: (0, 0)),
        ],
        out_specs=[
            pl.BlockSpec((_BN_PREP, 2 * EMB), lambda i: (i, 0)),
            pl.BlockSpec((_BN_PREP, 2 * EMB), lambda i: (i, 0)),
        ],
        out_shape=[
            jax.ShapeDtypeStruct((NUM_ITEMS, 2 * EMB), jnp.float32),
            jax.ShapeDtypeStruct((NUM_ITEMS, 2 * EMB), jnp.float32),
        ],
        compiler_params=pltpu.CompilerParams(
            dimension_semantics=("parallel",),
        ),
    )(ct, it, W1, b1r)


def _post_body(uw_ref, pw_ref, nw_ref, h_ref, w2_ref, b2_ref,
               u_ref, p_ref, n_ref, g_ref):
    u_ref[...] = uw_ref[:, :EMB]
    p_ref[...] = pw_ref[:, :EMB]
    n_ref[...] = nw_ref[:, :EMB]
    g_ref[...] = jnp.tanh(
        jnp.dot(h_ref[...], w2_ref[...], preferred_element_type=jnp.float32)
        + b2_ref[...]
    )


_BM_P = 2048


@jax.jit
def _tc_post(uw, pw, nw, h, W2, b2):
    b2r = b2.reshape(1, EMB)
    wide_spec = pl.BlockSpec((_BM_P, 2 * EMB), lambda i: (i, 0))
    out_spec = pl.BlockSpec((_BM_P, EMB), lambda i: (i, 0))
    return pl.pallas_call(
        _post_body,
        grid=(BATCH // _BM_P,),
        in_specs=[
            wide_spec, wide_spec, wide_spec, wide_spec,
            pl.BlockSpec((2 * EMB, EMB), lambda i: (0, 0)),
            pl.BlockSpec((1, EMB), lambda i: (0, 0)),
        ],
        out_specs=[out_spec, out_spec, out_spec, out_spec],
        out_shape=[
            jax.ShapeDtypeStruct((BATCH, EMB), jnp.float32),
            jax.ShapeDtypeStruct((BATCH, EMB), jnp.float32),
            jax.ShapeDtypeStruct((BATCH, EMB), jnp.float32),
            jax.ShapeDtypeStruct((BATCH, EMB), jnp.float32),
        ],
        compiler_params=pltpu.CompilerParams(
            dimension_semantics=("parallel",),
        ),
    )(uw, pw, nw, h, W2, b2r)


def kernel(uid, iid, nid, user_emb, item_emb, item_content, W1, b1, W2, b2):
    uid = uid.astype(jnp.int32)
    iid = iid.astype(jnp.int32)
    nid = nid.astype(jnp.int32)
    it = item_emb.T
    ct = item_content.T
    ut = user_emb.T
    htab, irm = _tc_prep(ct, it, W1, b1)
    uw = _sc_gatheru(uid, ut)
    pw = _sc_gather_row(iid, irm)
    nw = _sc_gather_row(nid, irm)
    h = _sc_gather_row(iid, htab)
    u, pos, neg, gen = _tc_post(uw, pw, nw, h, W2, b2)
    return (u, pos, neg, gen)


# speedup vs baseline: 3.3001x; 1.0017x over previous
import jax
import jax.numpy as jnp
from jax import lax
from jax.experimental import pallas as pl
from jax.experimental.pallas import tpu as pltpu
from jax.experimental.pallas import tpu_sc as plsc

NUM_USERS = 1000000
NUM_ITEMS = 100000
EMB = 64
CONTENT_DIM = 300
BATCH = 16384

_INFO = plsc.get_sparse_core_info()
_NC, _NS = _INFO.num_cores, _INFO.num_subcores
_NW = _NC * _NS
_ROWS_PER_W = BATCH // _NW
_UCHUNK = 256
_NUCHUNK = _ROWS_PER_W // _UCHUNK


_NBUF = 8
_UCHUNK = 128
_NUCH = _ROWS_PER_W // _UCHUNK


def _scalar_at(idx_v, j):
    vec = idx_v[pl.ds((j // 16) * 16, 16)]
    mask = lax.iota(jnp.int32, 16) == (j % 16)
    return jnp.sum(jnp.where(mask, vec, 0), axis=0)


def _sc_gatheru_body(uid_hbm, ut_hbm, u_out, idx_v, wbufs, out_v,
                     sem, out_sem):
    wid = lax.axis_index("s") * _NC + lax.axis_index("c")

    def chunk(i, _):
        base = wid * _ROWS_PER_W + i * _UCHUNK
        sl = pl.ds(base, _UCHUNK)
        pltpu.sync_copy(uid_hbm.at[sl], idx_v)
        for g in range(_UCHUNK // _NBUF):
            cps = []
            for k in range(_NBUF):
                j = g * _NBUF + k
                c = _scalar_at(idx_v, j)
                win = lax.shift_right_logical(c, 7)
                off = pl.multiple_of(win * 128, 128)
                cps.append(pltpu.async_copy(
                    ut_hbm.at[:, pl.ds(off, 128)], wbufs.at[k], sem))
            for cp in cps:
                cp.wait()
            for k in range(_NBUF):
                j = g * _NBUF + k
                c = _scalar_at(idx_v, j)
                lane = lax.bitwise_and(c, 127)
                col_idx = jnp.full((16,), lane, jnp.int32)
                slot_idx = jnp.full((16,), k, jnp.int32)
                for m in range(EMB // 16):
                    row_idx = lax.iota(jnp.int32, 16) + (16 * m)
                    gvals = plsc.load_gather(
                        wbufs, [slot_idx, row_idx, col_idx])
                    out_v[j, pl.ds(16 * m, 16)] = gvals
        pltpu.async_copy(out_v, u_out.at[sl], out_sem).wait()

    lax.fori_loop(0, _NUCH, chunk, None)


@jax.jit
def _sc_gatheru(uid, ut):
    mesh = plsc.VectorSubcoreMesh(core_axis_name="c", subcore_axis_name="s")
    fn = pl.kernel(
        _sc_gatheru_body,
        mesh=mesh,
        out_type=jax.ShapeDtypeStruct((BATCH, 2 * EMB), jnp.float32),
        scratch_types=[
            pltpu.VMEM((_UCHUNK,), jnp.int32),
            pltpu.VMEM((_NBUF, EMB, 2 * EMB), jnp.float32),
            pltpu.VMEM((_UCHUNK, 2 * EMB), jnp.float32),
            pltpu.SemaphoreType.DMA,
            pltpu.SemaphoreType.DMA,
        ],
        compiler_params=pltpu.CompilerParams(needs_layout_passes=False),
        cost_estimate=pl.CostEstimate(
            flops=0, bytes_accessed=540_000_000, transcendentals=0),
    )
    return fn(uid, ut)


_RCHUNK = 512


def _sc_gather_row_body(idx_hbm, tab_hbm, out_hbm, idx_v, row_v, sem):
    wid = lax.axis_index("s") * _NC + lax.axis_index("c")
    base = wid * _ROWS_PER_W
    sl = pl.ds(base, _RCHUNK)
    pltpu.sync_copy(idx_hbm.at[sl], idx_v)
    pltpu.async_copy(tab_hbm.at[idx_v], row_v, sem).wait()
    pltpu.sync_copy(row_v, out_hbm.at[sl])


@jax.jit
def _sc_gather_row(idx, tab):
    mesh = plsc.VectorSubcoreMesh(core_axis_name="c", subcore_axis_name="s")
    fn = pl.kernel(
        _sc_gather_row_body,
        mesh=mesh,
        out_type=jax.ShapeDtypeStruct((BATCH, 2 * EMB), jnp.float32),
        scratch_types=[
            pltpu.VMEM((_RCHUNK,), jnp.int32),
            pltpu.VMEM((_RCHUNK, 2 * EMB), jnp.float32),
            pltpu.SemaphoreType.DMA,
        ],
        cost_estimate=pl.CostEstimate(
            flops=0, bytes_accessed=18_000_000, transcendentals=0),
    )
    return fn(idx, tab)


def _prep_body(ct_ref, it_ref, w1_ref, b1_ref, htab_ref, irm_ref):
    h = lax.dot_general(
        ct_ref[...], w1_ref[...],
        dimension_numbers=(((0,), (0,)), ((), ())),
        preferred_element_type=jnp.float32,
    )
    htab_ref[...] = jnp.tanh(h + b1_ref[...])
    t = jnp.swapaxes(it_ref[...], 0, 1)
    irm_ref[...] = jnp.concatenate(
        [t, jnp.zeros_like(t)], axis=1)


_BN_PREP = 2048


@jax.jit
def _tc_prep(ct, it, W1, b1):
    b1r = b1.reshape(1, 2 * EMB)
    grid = (NUM_ITEMS + _BN_PREP - 1) // _BN_PREP
    return pl.pallas_call(
        _prep_body,
        grid=(grid,),
        in_specs=[
            pl.BlockSpec((CONTENT_DIM, _BN_PREP), lambda i: (0, i)),
            pl.BlockSpec((EMB, _BN_PREP), lambda i: (0, i)),
            pl.BlockSpec((CONTENT_DIM, 2 * EMB), lambda i: (0, 0)),
            pl.BlockSpec((1, 2 * EMB), lambda i: (0, 0)),
        ],
        out_specs=[
            pl.BlockSpec((_BN_PREP, 2 * EMB), lambda i: (i, 0)),
            pl.BlockSpec((_BN_PREP, 2 * EMB), lambda i: (i, 0)),
        ],
        out_shape=[
            jax.ShapeDtypeStruct((NUM_ITEMS, 2 * EMB), jnp.float32),
            jax.ShapeDtypeStruct((NUM_ITEMS, 2 * EMB), jnp.float32),
        ],
        compiler_params=pltpu.CompilerParams(
            dimension_semantics=("parallel",),
        ),
    )(ct, it, W1, b1r)


def _post_body(uw_ref, pw_ref, nw_ref, h_ref, w2_ref, b2_ref,
               u_ref, p_ref, n_ref, g_ref):
    u_ref[...] = uw_ref[:, :EMB]
    p_ref[...] = pw_ref[:, :EMB]
    n_ref[...] = nw_ref[:, :EMB]
    g_ref[...] = jnp.tanh(
        jnp.dot(h_ref[...], w2_ref[...], preferred_element_type=jnp.float32)
        + b2_ref[...]
    )


_BM_P = 2048


@jax.jit
def _tc_post(uw, pw, nw, h, W2, b2):
    b2r = b2.reshape(1, EMB)
    wide_spec = pl.BlockSpec((_BM_P, 2 * EMB), lambda i: (i, 0))
    out_spec = pl.BlockSpec((_BM_P, EMB), lambda i: (i, 0))
    return pl.pallas_call(
        _post_body,
        grid=(BATCH // _BM_P,),
        in_specs=[
            wide_spec, wide_spec, wide_spec, wide_spec,
            pl.BlockSpec((2 * EMB, EMB), lambda i: (0, 0)),
            pl.BlockSpec((1, EMB), lambda i: (0, 0)),
        ],
        out_specs=[out_spec, out_spec, out_spec, out_spec],
        out_shape=[
            jax.ShapeDtypeStruct((BATCH, EMB), jnp.float32),
            jax.ShapeDtypeStruct((BATCH, EMB), jnp.float32),
            jax.ShapeDtypeStruct((BATCH, EMB), jnp.float32),
            jax.ShapeDtypeStruct((BATCH, EMB), jnp.float32),
        ],
        compiler_params=pltpu.CompilerParams(
            dimension_semantics=("parallel",),
        ),
    )(uw, pw, nw, h, W2, b2r)


def kernel(uid, iid, nid, user_emb, item_emb, item_content, W1, b1, W2, b2):
    uid = uid.astype(jnp.int32)
    iid = iid.astype(jnp.int32)
    nid = nid.astype(jnp.int32)
    it = item_emb.T
    ct = item_content.T
    ut = user_emb.T
    uw = _sc_gatheru(uid, ut)
    htab, irm = _tc_prep(ct, it, W1, b1)
    pw = _sc_gather_row(iid, irm)
    nw = _sc_gather_row(nid, irm)
    h = _sc_gather_row(iid, htab)
    u, pos, neg, gen = _tc_post(uw, pw, nw, h, W2, b2)
    return (u, pos, neg, gen)


# speedup vs baseline: 3.6522x; 1.1067x over previous
import jax
import jax.numpy as jnp
from jax import lax
from jax.experimental import pallas as pl
from jax.experimental.pallas import tpu as pltpu
from jax.experimental.pallas import tpu_sc as plsc

NUM_USERS = 1000000
NUM_ITEMS = 100000
EMB = 64
CONTENT_DIM = 300
BATCH = 16384

_INFO = plsc.get_sparse_core_info()
_NC, _NS = _INFO.num_cores, _INFO.num_subcores
_NW = _NC * _NS
_ROWS_PER_W = BATCH // _NW
_UCHUNK = 256
_NUCHUNK = _ROWS_PER_W // _UCHUNK


_NBUF = 8
_UCHUNK = 128
_NUCH = _ROWS_PER_W // _UCHUNK


def _scalar_at(idx_v, j):
    vec = idx_v[pl.ds((j // 16) * 16, 16)]
    mask = lax.iota(jnp.int32, 16) == (j % 16)
    return jnp.sum(jnp.where(mask, vec, 0), axis=0)


def _sc_gatheru_body(uid_hbm, ut_hbm, u_out, idx_v, wbufs, out_v,
                     sem, out_sem):
    wid = lax.axis_index("s") * _NC + lax.axis_index("c")

    def chunk(i, _):
        base = wid * _ROWS_PER_W + i * _UCHUNK
        sl = pl.ds(base, _UCHUNK)
        pltpu.sync_copy(uid_hbm.at[sl], idx_v)
        for g in range(_UCHUNK // _NBUF):
            cps = []
            for k in range(_NBUF):
                j = g * _NBUF + k
                c = _scalar_at(idx_v, j)
                win = lax.shift_right_logical(c, 7)
                off = pl.multiple_of(win * 128, 128)
                cps.append(pltpu.async_copy(
                    ut_hbm.at[:, pl.ds(off, 128)], wbufs.at[k], sem))
            for cp in cps:
                cp.wait()
            for k in range(_NBUF):
                j = g * _NBUF + k
                c = _scalar_at(idx_v, j)
                lane = lax.bitwise_and(c, 127)
                col_idx = jnp.full((16,), lane, jnp.int32)
                slot_idx = jnp.full((16,), k, jnp.int32)
                for m in range(EMB // 16):
                    row_idx = lax.iota(jnp.int32, 16) + (16 * m)
                    gvals = plsc.load_gather(
                        wbufs, [slot_idx, row_idx, col_idx])
                    out_v[j, pl.ds(16 * m, 16)] = gvals
        pltpu.async_copy(out_v, u_out.at[sl], out_sem).wait()

    lax.fori_loop(0, _NUCH, chunk, None)


@jax.jit
def _sc_gatheru(uid, ut):
    mesh = plsc.VectorSubcoreMesh(core_axis_name="c", subcore_axis_name="s")
    fn = pl.kernel(
        _sc_gatheru_body,
        mesh=mesh,
        out_type=jax.ShapeDtypeStruct((BATCH, 2 * EMB), jnp.float32),
        scratch_types=[
            pltpu.VMEM((_UCHUNK,), jnp.int32),
            pltpu.VMEM((_NBUF, EMB, 2 * EMB), jnp.float32),
            pltpu.VMEM((_UCHUNK, 2 * EMB), jnp.float32),
            pltpu.SemaphoreType.DMA,
            pltpu.SemaphoreType.DMA,
        ],
        compiler_params=pltpu.CompilerParams(needs_layout_passes=False),
        cost_estimate=pl.CostEstimate(
            flops=0, bytes_accessed=540_000_000, transcendentals=0),
    )
    return fn(uid, ut)


_RCHUNK = 512


def _sc_gather_row_body(idx_hbm, tab_hbm, out_hbm, idx_v, row_v, sem):
    wid = lax.axis_index("s") * _NC + lax.axis_index("c")
    base = wid * _ROWS_PER_W
    sl = pl.ds(base, _RCHUNK)
    pltpu.sync_copy(idx_hbm.at[sl], idx_v)
    pltpu.async_copy(tab_hbm.at[idx_v], row_v, sem).wait()
    pltpu.sync_copy(row_v, out_hbm.at[sl])


@jax.jit
def _sc_gather_row(idx, tab):
    mesh = plsc.VectorSubcoreMesh(core_axis_name="c", subcore_axis_name="s")
    fn = pl.kernel(
        _sc_gather_row_body,
        mesh=mesh,
        out_type=jax.ShapeDtypeStruct((BATCH, 2 * EMB), jnp.float32),
        scratch_types=[
            pltpu.VMEM((_RCHUNK,), jnp.int32),
            pltpu.VMEM((_RCHUNK, 2 * EMB), jnp.float32),
            pltpu.SemaphoreType.DMA,
        ],
        cost_estimate=pl.CostEstimate(
            flops=0, bytes_accessed=18_000_000, transcendentals=0),
    )
    return fn(idx, tab)


def _prep_body(ct_ref, it_ref, w1_ref, b1_ref, htab_ref, irm_ref):
    h = lax.dot_general(
        ct_ref[...], w1_ref[...],
        dimension_numbers=(((0,), (0,)), ((), ())),
        preferred_element_type=jnp.float32,
    )
    htab_ref[...] = jnp.tanh(h + b1_ref[...])
    t = jnp.swapaxes(it_ref[...], 0, 1)
    irm_ref[...] = jnp.concatenate(
        [t, jnp.zeros_like(t)], axis=1)


_BN_PREP = 2048


@jax.jit
def _tc_prep(ct, it, W1, b1):
    b1r = b1.reshape(1, 2 * EMB)
    grid = (NUM_ITEMS + _BN_PREP - 1) // _BN_PREP
    return pl.pallas_call(
        _prep_body,
        grid=(grid,),
        in_specs=[
            pl.BlockSpec((CONTENT_DIM, _BN_PREP), lambda i: (0, i)),
            pl.BlockSpec((EMB, _BN_PREP), lambda i: (0, i)),
            pl.BlockSpec((CONTENT_DIM, 2 * EMB), lambda i: (0, 0)),
            pl.BlockSpec((1, 2 * EMB), lambda i: (0, 0)),
        ],
        out_specs=[
            pl.BlockSpec((_BN_PREP, 2 * EMB), lambda i: (i, 0)),
            pl.BlockSpec((_BN_PREP, 2 * EMB), lambda i: (i, 0)),
        ],
        out_shape=[
            jax.ShapeDtypeStruct((NUM_ITEMS, 2 * EMB), jnp.float32),
            jax.ShapeDtypeStruct((NUM_ITEMS, 2 * EMB), jnp.float32),
        ],
        compiler_params=pltpu.CompilerParams(
            dimension_semantics=("parallel",),
        ),
    )(ct, it, W1, b1r)


def _post_body(uw_ref, pw_ref, nw_ref, h_ref, w2_ref, b2_ref,
               u_ref, p_ref, n_ref, g_ref):
    u_ref[...] = uw_ref[:, :EMB]
    p_ref[...] = pw_ref[:, :EMB]
    n_ref[...] = nw_ref[:, :EMB]
    g_ref[...] = jnp.tanh(
        jnp.dot(h_ref[...], w2_ref[...], preferred_element_type=jnp.float32)
        + b2_ref[...]
    )


_BM_P = 2048


@jax.jit
def _tc_post(uw, pw, nw, h, W2, b2):
    b2r = b2.reshape(1, EMB)
    wide_spec = pl.BlockSpec((_BM_P, 2 * EMB), lambda i: (i, 0))
    out_spec = pl.BlockSpec((_BM_P, EMB), lambda i: (i, 0))
    return pl.pallas_call(
        _post_body,
        grid=(BATCH // _BM_P,),
        in_specs=[
            wide_spec, wide_spec, wide_spec, wide_spec,
            pl.BlockSpec((2 * EMB, EMB), lambda i: (0, 0)),
            pl.BlockSpec((1, EMB), lambda i: (0, 0)),
        ],
        out_specs=[out_spec, out_spec, out_spec, out_spec],
        out_shape=[
            jax.ShapeDtypeStruct((BATCH, EMB), jnp.float32),
            jax.ShapeDtypeStruct((BATCH, EMB), jnp.float32),
            jax.ShapeDtypeStruct((BATCH, EMB), jnp.float32),
            jax.ShapeDtypeStruct((BATCH, EMB), jnp.float32),
        ],
        compiler_params=pltpu.CompilerParams(
            dimension_semantics=("parallel",),
        ),
    )(uw, pw, nw, h, W2, b2r)


def kernel(uid, iid, nid, user_emb, item_emb, item_content, W1, b1, W2, b2):
    uid = uid.astype(jnp.int32)
    iid = iid.astype(jnp.int32)
    nid = nid.astype(jnp.int32)
    it = item_emb.T
    ct = item_content.T
    ut = user_emb.T
    uw = _sc_gatheru(uid, ut)
    htab, irm = _tc_prep(ct, it, W1, b1)
    iid, nid, uw = lax.optimization_barrier((iid, nid, uw))
    pw = _sc_gather_row(iid, irm)
    nw = _sc_gather_row(nid, irm)
    h = _sc_gather_row(iid, htab)
    u, pos, neg, gen = _tc_post(uw, pw, nw, h, W2, b2)
    return (u, pos, neg, gen)


# speedup vs baseline: 3.9757x; 1.0886x over previous
import jax
import jax.numpy as jnp
from jax import lax
from jax.experimental import pallas as pl
from jax.experimental.pallas import tpu as pltpu
from jax.experimental.pallas import tpu_sc as plsc

NUM_USERS = 1000000
NUM_ITEMS = 100000
EMB = 64
CONTENT_DIM = 300
BATCH = 16384

_INFO = plsc.get_sparse_core_info()
_NC, _NS = _INFO.num_cores, _INFO.num_subcores
_NW = _NC * _NS
_ROWS_PER_W = BATCH // _NW
_UCHUNK = 256
_NUCHUNK = _ROWS_PER_W // _UCHUNK


_NBUF = 8
_UCHUNK = 128
_NUCH = _ROWS_PER_W // _UCHUNK


def _scalar_at(idx_v, j):
    vec = idx_v[pl.ds((j // 16) * 16, 16)]
    mask = lax.iota(jnp.int32, 16) == (j % 16)
    return jnp.sum(jnp.where(mask, vec, 0), axis=0)


def _sc_gatheru_body(uid_hbm, ut_hbm, u_out, idx_v, wbufs, out_v,
                     sem, out_sem):
    wid = lax.axis_index("s") * _NC + lax.axis_index("c")

    def chunk(i, _):
        base = wid * _ROWS_PER_W + i * _UCHUNK
        sl = pl.ds(base, _UCHUNK)
        pltpu.sync_copy(uid_hbm.at[sl], idx_v)
        for g in range(_UCHUNK // _NBUF):
            cps = []
            for k in range(_NBUF):
                j = g * _NBUF + k
                c = _scalar_at(idx_v, j)
                win = lax.shift_right_logical(c, 7)
                off = pl.multiple_of(win * 128, 128)
                cps.append(pltpu.async_copy(
                    ut_hbm.at[:, pl.ds(off, 128)], wbufs.at[k], sem))
            for cp in cps:
                cp.wait()
            for k in range(_NBUF):
                j = g * _NBUF + k
                c = _scalar_at(idx_v, j)
                lane = lax.bitwise_and(c, 127)
                col_idx = jnp.full((16,), lane, jnp.int32)
                slot_idx = jnp.full((16,), k, jnp.int32)
                for m in range(EMB // 16):
                    row_idx = lax.iota(jnp.int32, 16) + (16 * m)
                    gvals = plsc.load_gather(
                        wbufs, [slot_idx, row_idx, col_idx])
                    out_v[j, pl.ds(16 * m, 16)] = gvals
        pltpu.async_copy(out_v, u_out.at[sl], out_sem).wait()

    lax.fori_loop(0, _NUCH, chunk, None)


@jax.jit
def _sc_gatheru(uid, ut):
    mesh = plsc.VectorSubcoreMesh(core_axis_name="c", subcore_axis_name="s")
    fn = pl.kernel(
        _sc_gatheru_body,
        mesh=mesh,
        out_type=jax.ShapeDtypeStruct((BATCH, 2 * EMB), jnp.float32),
        scratch_types=[
            pltpu.VMEM((_UCHUNK,), jnp.int32),
            pltpu.VMEM((_NBUF, EMB, 2 * EMB), jnp.float32),
            pltpu.VMEM((_UCHUNK, 2 * EMB), jnp.float32),
            pltpu.SemaphoreType.DMA,
            pltpu.SemaphoreType.DMA,
        ],
        compiler_params=pltpu.CompilerParams(needs_layout_passes=False),
        cost_estimate=pl.CostEstimate(
            flops=0, bytes_accessed=540_000_000, transcendentals=0),
    )
    return fn(uid, ut)


_RCHUNK = 512


def _sc_gather_row_body(idx_hbm, tab_hbm, out_hbm, idx_v, row_v, sem):
    wid = lax.axis_index("s") * _NC + lax.axis_index("c")
    base = wid * _ROWS_PER_W
    sl = pl.ds(base, _RCHUNK)
    pltpu.sync_copy(idx_hbm.at[sl], idx_v)
    pltpu.async_copy(tab_hbm.at[idx_v], row_v, sem).wait()
    pltpu.sync_copy(row_v, out_hbm.at[sl])


@jax.jit
def _sc_gather_row(idx, tab):
    mesh = plsc.VectorSubcoreMesh(core_axis_name="c", subcore_axis_name="s")
    fn = pl.kernel(
        _sc_gather_row_body,
        mesh=mesh,
        out_type=jax.ShapeDtypeStruct((BATCH, 2 * EMB), jnp.float32),
        scratch_types=[
            pltpu.VMEM((_RCHUNK,), jnp.int32),
            pltpu.VMEM((_RCHUNK, 2 * EMB), jnp.float32),
            pltpu.SemaphoreType.DMA,
        ],
        cost_estimate=pl.CostEstimate(
            flops=0, bytes_accessed=18_000_000, transcendentals=0),
    )
    return fn(idx, tab)


def _prep_body(ct_ref, it_ref, w1_ref, b1_ref, htab_ref, irm_ref):
    h = lax.dot_general(
        ct_ref[...], w1_ref[...],
        dimension_numbers=(((0,), (0,)), ((), ())),
        preferred_element_type=jnp.float32,
    )
    htab_ref[...] = jnp.tanh(h + b1_ref[...])
    t = jnp.swapaxes(it_ref[...], 0, 1)
    irm_ref[...] = jnp.concatenate(
        [t, jnp.zeros_like(t)], axis=1)


_BN_PREP = 2048


@jax.jit
def _tc_prep(ct, it, W1, b1):
    b1r = b1.reshape(1, 2 * EMB)
    grid = (NUM_ITEMS + _BN_PREP - 1) // _BN_PREP
    return pl.pallas_call(
        _prep_body,
        grid=(grid,),
        in_specs=[
            pl.BlockSpec((CONTENT_DIM, _BN_PREP), lambda i: (0, i)),
            pl.BlockSpec((EMB, _BN_PREP), lambda i: (0, i)),
            pl.BlockSpec((CONTENT_DIM, 2 * EMB), lambda i: (0, 0)),
            pl.BlockSpec((1, 2 * EMB), lambda i: (0, 0)),
        ],
        out_specs=[
            pl.BlockSpec((_BN_PREP, 2 * EMB), lambda i: (i, 0)),
            pl.BlockSpec((_BN_PREP, 2 * EMB), lambda i: (i, 0)),
        ],
        out_shape=[
            jax.ShapeDtypeStruct((NUM_ITEMS, 2 * EMB), jnp.float32),
            jax.ShapeDtypeStruct((NUM_ITEMS, 2 * EMB), jnp.float32),
        ],
        compiler_params=pltpu.CompilerParams(
            dimension_semantics=("parallel",),
        ),
    )(ct, it, W1, b1r)


def _post_body(uw_ref, pw_ref, nw_ref, h_ref, w2_ref, b2_ref,
               u_ref, p_ref, n_ref, g_ref):
    u_ref[...] = jnp.swapaxes(uw_ref[:, :EMB], 0, 1)
    p_ref[...] = jnp.swapaxes(pw_ref[:, :EMB], 0, 1)
    n_ref[...] = jnp.swapaxes(nw_ref[:, :EMB], 0, 1)
    g = lax.dot_general(
        w2_ref[...], h_ref[...],
        dimension_numbers=(((0,), (1,)), ((), ())),
        preferred_element_type=jnp.float32,
    )
    g_ref[...] = jnp.tanh(g + b2_ref[...])


_BM_P = 2048


@jax.jit
def _tc_post(uw, pw, nw, h, W2, b2):
    b2c = b2.reshape(EMB, 1)
    wide_spec = pl.BlockSpec((_BM_P, 2 * EMB), lambda i: (i, 0))
    out_spec = pl.BlockSpec((EMB, _BM_P), lambda i: (0, i))
    return pl.pallas_call(
        _post_body,
        grid=(BATCH // _BM_P,),
        in_specs=[
            wide_spec, wide_spec, wide_spec, wide_spec,
            pl.BlockSpec((2 * EMB, EMB), lambda i: (0, 0)),
            pl.BlockSpec((EMB, 1), lambda i: (0, 0)),
        ],
        out_specs=[out_spec, out_spec, out_spec, out_spec],
        out_shape=[
            jax.ShapeDtypeStruct((EMB, BATCH), jnp.float32),
            jax.ShapeDtypeStruct((EMB, BATCH), jnp.float32),
            jax.ShapeDtypeStruct((EMB, BATCH), jnp.float32),
            jax.ShapeDtypeStruct((EMB, BATCH), jnp.float32),
        ],
        compiler_params=pltpu.CompilerParams(
            dimension_semantics=("parallel",),
        ),
    )(uw, pw, nw, h, W2, b2c)


def kernel(uid, iid, nid, user_emb, item_emb, item_content, W1, b1, W2, b2):
    uid = uid.astype(jnp.int32)
    iid = iid.astype(jnp.int32)
    nid = nid.astype(jnp.int32)
    it = item_emb.T
    ct = item_content.T
    ut = user_emb.T
    uw = _sc_gatheru(uid, ut)
    htab, irm = _tc_prep(ct, it, W1, b1)
    iid, nid, uw = lax.optimization_barrier((iid, nid, uw))
    pw = _sc_gather_row(iid, irm)
    nw = _sc_gather_row(nid, irm)
    h = _sc_gather_row(iid, htab)
    u_t, pos_t, neg_t, gen_t = _tc_post(uw, pw, nw, h, W2, b2)
    return (u_t.T, pos_t.T, neg_t.T, gen_t.T)


# speedup vs baseline: 4.0354x; 1.0150x over previous
import jax
import jax.numpy as jnp
from jax import lax
from jax.experimental import pallas as pl
from jax.experimental.pallas import tpu as pltpu
from jax.experimental.pallas import tpu_sc as plsc

NUM_USERS = 1000000
NUM_ITEMS = 100000
EMB = 64
CONTENT_DIM = 300
BATCH = 16384

_INFO = plsc.get_sparse_core_info()
_NC, _NS = _INFO.num_cores, _INFO.num_subcores
_NW = _NC * _NS
_ROWS_PER_W = BATCH // _NW
_UCHUNK = 256
_NUCHUNK = _ROWS_PER_W // _UCHUNK


_NBUF = 8
_UCHUNK = 128
_NUCH = _ROWS_PER_W // _UCHUNK


def _scalar_at(idx_v, j):
    vec = idx_v[pl.ds((j // 16) * 16, 16)]
    mask = lax.iota(jnp.int32, 16) == (j % 16)
    return jnp.sum(jnp.where(mask, vec, 0), axis=0)


_HALF = _NBUF // 2


def _sc_gatheru_body(uid_hbm, ut_hbm, u_out, idx_v, wbufs, out_v,
                     sem_a, sem_b, out_sem):
    wid = lax.axis_index("s") * _NC + lax.axis_index("c")
    sems = (sem_a, sem_b)
    ngroups = _UCHUNK // _HALF

    def fire(g):
        half = g % 2
        cps = []
        for k in range(_HALF):
            j = g * _HALF + k
            c = _scalar_at(idx_v, j)
            win = lax.shift_right_logical(c, 7)
            off = pl.multiple_of(win * 128, 128)
            cps.append(pltpu.async_copy(
                ut_hbm.at[:, pl.ds(off, 128)], wbufs.at[half * _HALF + k],
                sems[half]))
        return cps

    def extract(g):
        half = g % 2
        for k in range(_HALF):
            j = g * _HALF + k
            c = _scalar_at(idx_v, j)
            lane = lax.bitwise_and(c, 127)
            col_idx = jnp.full((16,), lane, jnp.int32)
            slot_idx = jnp.full((16,), half * _HALF + k, jnp.int32)
            for m in range(EMB // 16):
                row_idx = lax.iota(jnp.int32, 16) + (16 * m)
                gvals = plsc.load_gather(wbufs, [slot_idx, row_idx, col_idx])
                out_v[j, pl.ds(16 * m, 16)] = gvals

    def chunk(i, _):
        base = wid * _ROWS_PER_W + i * _UCHUNK
        sl = pl.ds(base, _UCHUNK)
        pltpu.sync_copy(uid_hbm.at[sl], idx_v)
        prev = fire(0)
        for g in range(1, ngroups):
            cur = fire(g)
            for cp in prev:
                cp.wait()
            extract(g - 1)
            prev = cur
        for cp in prev:
            cp.wait()
        extract(ngroups - 1)
        pltpu.async_copy(out_v, u_out.at[sl], out_sem).wait()

    lax.fori_loop(0, _NUCH, chunk, None)


@jax.jit
def _sc_gatheru(uid, ut):
    mesh = plsc.VectorSubcoreMesh(core_axis_name="c", subcore_axis_name="s")
    fn = pl.kernel(
        _sc_gatheru_body,
        mesh=mesh,
        out_type=jax.ShapeDtypeStruct((BATCH, 2 * EMB), jnp.float32),
        scratch_types=[
            pltpu.VMEM((_UCHUNK,), jnp.int32),
            pltpu.VMEM((_NBUF, EMB, 2 * EMB), jnp.float32),
            pltpu.VMEM((_UCHUNK, 2 * EMB), jnp.float32),
            pltpu.SemaphoreType.DMA,
            pltpu.SemaphoreType.DMA,
            pltpu.SemaphoreType.DMA,
        ],
        compiler_params=pltpu.CompilerParams(needs_layout_passes=False),
        cost_estimate=pl.CostEstimate(
            flops=0, bytes_accessed=540_000_000, transcendentals=0),
    )
    return fn(uid, ut)


_RCHUNK = 512


def _sc_gather_row_body(idx_hbm, tab_hbm, out_hbm, idx_v, row_v, sem):
    wid = lax.axis_index("s") * _NC + lax.axis_index("c")
    base = wid * _ROWS_PER_W
    sl = pl.ds(base, _RCHUNK)
    pltpu.sync_copy(idx_hbm.at[sl], idx_v)
    pltpu.async_copy(tab_hbm.at[idx_v], row_v, sem).wait()
    pltpu.sync_copy(row_v, out_hbm.at[sl])


@jax.jit
def _sc_gather_row(idx, tab):
    mesh = plsc.VectorSubcoreMesh(core_axis_name="c", subcore_axis_name="s")
    fn = pl.kernel(
        _sc_gather_row_body,
        mesh=mesh,
        out_type=jax.ShapeDtypeStruct((BATCH, 2 * EMB), jnp.float32),
        scratch_types=[
            pltpu.VMEM((_RCHUNK,), jnp.int32),
            pltpu.VMEM((_RCHUNK, 2 * EMB), jnp.float32),
            pltpu.SemaphoreType.DMA,
        ],
        cost_estimate=pl.CostEstimate(
            flops=0, bytes_accessed=18_000_000, transcendentals=0),
    )
    return fn(idx, tab)


def _prep_body(ct_ref, it_ref, w1_ref, b1_ref, htab_ref, irm_ref):
    h = lax.dot_general(
        ct_ref[...], w1_ref[...],
        dimension_numbers=(((0,), (0,)), ((), ())),
        preferred_element_type=jnp.float32,
    )
    htab_ref[...] = jnp.tanh(h + b1_ref[...])
    t = jnp.swapaxes(it_ref[...], 0, 1)
    irm_ref[...] = jnp.concatenate(
        [t, jnp.zeros_like(t)], axis=1)


_BN_PREP = 2048


@jax.jit
def _tc_prep(ct, it, W1, b1):
    b1r = b1.reshape(1, 2 * EMB)
    grid = (NUM_ITEMS + _BN_PREP - 1) // _BN_PREP
    return pl.pallas_call(
        _prep_body,
        grid=(grid,),
        in_specs=[
            pl.BlockSpec((CONTENT_DIM, _BN_PREP), lambda i: (0, i)),
            pl.BlockSpec((EMB, _BN_PREP), lambda i: (0, i)),
            pl.BlockSpec((CONTENT_DIM, 2 * EMB), lambda i: (0, 0)),
            pl.BlockSpec((1, 2 * EMB), lambda i: (0, 0)),
        ],
        out_specs=[
            pl.BlockSpec((_BN_PREP, 2 * EMB), lambda i: (i, 0)),
            pl.BlockSpec((_BN_PREP, 2 * EMB), lambda i: (i, 0)),
        ],
        out_shape=[
            jax.ShapeDtypeStruct((NUM_ITEMS, 2 * EMB), jnp.float32),
            jax.ShapeDtypeStruct((NUM_ITEMS, 2 * EMB), jnp.float32),
        ],
        compiler_params=pltpu.CompilerParams(
            dimension_semantics=("parallel",),
        ),
    )(ct, it, W1, b1r)


def _post_body(uw_ref, pw_ref, nw_ref, h_ref, w2_ref, b2_ref,
               u_ref, p_ref, n_ref, g_ref):
    u_ref[...] = jnp.swapaxes(uw_ref[:, :EMB], 0, 1)
    p_ref[...] = jnp.swapaxes(pw_ref[:, :EMB], 0, 1)
    n_ref[...] = jnp.swapaxes(nw_ref[:, :EMB], 0, 1)
    g = lax.dot_general(
        w2_ref[...], h_ref[...],
        dimension_numbers=(((0,), (1,)), ((), ())),
        preferred_element_type=jnp.float32,
    )
    g_ref[...] = jnp.tanh(g + b2_ref[...])


_BM_P = 2048


@jax.jit
def _tc_post(uw, pw, nw, h, W2, b2):
    b2c = b2.reshape(EMB, 1)
    wide_spec = pl.BlockSpec((_BM_P, 2 * EMB), lambda i: (i, 0))
    out_spec = pl.BlockSpec((EMB, _BM_P), lambda i: (0, i))
    return pl.pallas_call(
        _post_body,
        grid=(BATCH // _BM_P,),
        in_specs=[
            wide_spec, wide_spec, wide_spec, wide_spec,
            pl.BlockSpec((2 * EMB, EMB), lambda i: (0, 0)),
            pl.BlockSpec((EMB, 1), lambda i: (0, 0)),
        ],
        out_specs=[out_spec, out_spec, out_spec, out_spec],
        out_shape=[
            jax.ShapeDtypeStruct((EMB, BATCH), jnp.float32),
            jax.ShapeDtypeStruct((EMB, BATCH), jnp.float32),
            jax.ShapeDtypeStruct((EMB, BATCH), jnp.float32),
            jax.ShapeDtypeStruct((EMB, BATCH), jnp.float32),
        ],
        compiler_params=pltpu.CompilerParams(
            dimension_semantics=("parallel",),
        ),
    )(uw, pw, nw, h, W2, b2c)


def kernel(uid, iid, nid, user_emb, item_emb, item_content, W1, b1, W2, b2):
    uid = uid.astype(jnp.int32)
    iid = iid.astype(jnp.int32)
    nid = nid.astype(jnp.int32)
    it = item_emb.T
    ct = item_content.T
    ut = user_emb.T
    uw = _sc_gatheru(uid, ut)
    htab, irm = _tc_prep(ct, it, W1, b1)
    iid, nid, uw = lax.optimization_barrier((iid, nid, uw))
    pw = _sc_gather_row(iid, irm)
    nw = _sc_gather_row(nid, irm)
    h = _sc_gather_row(iid, htab)
    u_t, pos_t, neg_t, gen_t = _tc_post(uw, pw, nw, h, W2, b2)
    return (u_t.T, pos_t.T, neg_t.T, gen_t.T)
